# baseline (speedup 1.0000x reference)
import functools

import jax
import jax.numpy as jnp
from jax import lax
from jax.experimental import pallas as pl
from jax.experimental.pallas import tpu as pltpu
from jax.experimental.pallas import tpu_sc as plsc

N = 10000
E = 160000
D = 256
DC = 128

NC, NS = 2, 16
NP = 10112
EP = 163840
G = 128
NBUF = 3
NRING = 4
NB = EP // G
BA = NB // NS
GD = 128
ND = EP // GD
BD = ND // (NC * NS)
ZR = NP // NS
PAD_ROW = N + 100
PAD_COL = N

_CP = pltpu.CompilerParams(needs_layout_passes=False)


def _sc_mesh():
    return plsc.VectorSubcoreMesh(core_axis_name="c", subcore_axis_name="s")


def _sc_degree(rows2d):
    @functools.partial(
        pl.kernel,
        out_type=jax.ShapeDtypeStruct((NC, NP), jnp.float32),
        mesh=_sc_mesh(),
        compiler_params=_CP,
        scratch_types=[
            pltpu.VMEM((BD, GD), jnp.int32),
            pltpu.VMEM((GD,), jnp.float32),
            pltpu.VMEM((128,), jnp.float32),
            pltpu.VMEM_SHARED((NP,), jnp.float32),
        ],
    )
    def deg_kernel(rows_hbm, out_hbm, erow, ones, zeros, acc):
        c = lax.axis_index("c")
        s = lax.axis_index("s")
        ones16 = jnp.ones((16,), jnp.float32)
        zeros16 = jnp.zeros((16,), jnp.float32)

        @pl.loop(0, GD // 16)
        def _(i):
            ones[pl.ds(i * 16, 16)] = ones16

        @pl.loop(0, 128 // 16)
        def _(i):
            zeros[pl.ds(i * 16, 16)] = zeros16

        pltpu.sync_copy(rows_hbm.at[pl.ds((c * NS + s) * BD, BD)], erow)

        @pl.loop(0, (NP // 128 + NS - 1) // NS)
        def _(i):
            blk = i * NS + s

            @pl.when(blk < NP // 128)
            def _():
                pltpu.sync_copy(zeros, acc.at[pl.ds(blk * 128, 128)])

        plsc.subcore_barrier()

        @pl.loop(0, BD)
        def _(j):
            pltpu.sync_copy(ones, acc.at[erow.at[j]], add=True)

        plsc.subcore_barrier()

        @pl.loop(0, (NP // 128 + NS - 1) // NS)
        def _(i):
            blk = i * NS + s

            @pl.when(blk < NP // 128)
            def _():
                pltpu.sync_copy(acc.at[pl.ds(blk * 128, 128)],
                                out_hbm.at[c].at[pl.ds(blk * 128, 128)])

    return deg_kernel(rows2d)


def _norm_body(x_ref, hist_ref, g_ref, bt_ref, hs_ref, dis_ref, *, bm):
    i = pl.program_id(0)
    xb = x_ref[...]
    mu = jnp.mean(xb, axis=1, keepdims=True)
    xc = xb - mu
    var = jnp.mean(xc * xc, axis=1, keepdims=True)
    h = xc * lax.rsqrt(var + 1e-5) * g_ref[...] + bt_ref[...]
    deg = jnp.sum(hist_ref[...], axis=1, keepdims=True) + 1.0
    dis = lax.rsqrt(deg)
    row = i * bm + lax.broadcasted_iota(jnp.int32, (bm, 1), 0)
    hs = jnp.where(row < N, h * dis, 0.0)
    hs_ref[0] = hs[:, :DC]
    hs_ref[1] = hs[:, DC:]
    dis_ref[...] = dis


def _tc_norm_scale(x_pad, hist_t, gamma2, beta2):
    bm = ZR
    return pl.pallas_call(
        functools.partial(_norm_body, bm=bm),
        grid=(NP // bm,),
        in_specs=[
            pl.BlockSpec((bm, D), lambda i: (i, 0)),
            pl.BlockSpec((bm, NC), lambda i: (i, 0)),
            pl.BlockSpec((1, D), lambda i: (0, 0)),
            pl.BlockSpec((1, D), lambda i: (0, 0)),
        ],
        out_specs=[
            pl.BlockSpec((NC, bm, DC), lambda i: (0, i, 0)),
            pl.BlockSpec((bm, 1), lambda i: (i, 0)),
        ],
        out_shape=[
            jax.ShapeDtypeStruct((NC, NP, DC), jnp.float32),
            jax.ShapeDtypeStruct((NP, 1), jnp.float32),
        ],
    )(x_pad, hist_t, gamma2, beta2)


def _sc_aggregate(eidx, hs2):
    @functools.partial(
        pl.kernel,
        out_type=jax.ShapeDtypeStruct((NC, NP, DC), jnp.float32),
        mesh=_sc_mesh(),
        compiler_params=_CP,
        scratch_types=[
            pltpu.VMEM((NRING, 2, G), jnp.int32),
            pltpu.VMEM((NBUF, G, DC), jnp.float32),
            pltpu.VMEM_SHARED((NP, DC), jnp.float32),
            pltpu.SemaphoreType.DMA,
            pltpu.SemaphoreType.DMA,
            pltpu.SemaphoreType.DMA,
        ],
    )
    def agg_kernel(eidx_hbm, hs_hbm, out_hbm, iring, buf, acc,
                   isem, gsem, ssem):
        c = lax.axis_index("c")
        s = lax.axis_index("s")

        pltpu.sync_copy(hs_hbm.at[c].at[pl.ds(s * ZR, ZR)],
                        acc.at[pl.ds(s * ZR, ZR)])
        plsc.subcore_barrier()

        def idx_fetch(j):
            pltpu.async_copy(
                eidx_hbm.at[s * BA + j], iring.at[j % NRING], isem)

        def idx_wait(j):
            pltpu.make_async_copy(
                eidx_hbm.at[s * BA + j], iring.at[j % NRING], isem).wait()

        def gather(j):
            pltpu.async_copy(hs_hbm.at[c].at[iring.at[j % NRING].at[1]],
                             buf.at[j % NBUF], gsem)

        def gather_wait(j):
            pltpu.make_async_copy(hs_hbm.at[c].at[iring.at[j % NRING].at[1]],
                                  buf.at[j % NBUF], gsem).wait()

        def scatter(j):
            pltpu.async_copy(buf.at[j % NBUF],
                             acc.at[iring.at[j % NRING].at[0]], ssem,
                             add=True)

        def scatter_wait(j):
            pltpu.make_async_copy(buf.at[j % NBUF],
                                  acc.at[iring.at[j % NRING].at[0]],
                                  ssem).wait()

        for k in range(NRING - 1):
            idx_fetch(k)
        for k in range(NBUF - 1):
            idx_wait(k)
            gather(k)

        @pl.loop(0, BA)
        def _(j):
            @pl.when(j >= 1)
            def _():
                scatter_wait(j - 1)

            @pl.when(j + NRING - 1 < BA)
            def _():
                idx_fetch(j + NRING - 1)

            @pl.when(j + NBUF - 1 < BA)
            def _():
                idx_wait(j + NBUF - 1)
                gather(j + NBUF - 1)

            gather_wait(j)
            scatter(j)

        scatter_wait(BA - 1)
        plsc.subcore_barrier()
        pltpu.sync_copy(acc.at[pl.ds(s * ZR, ZR)],
                        out_hbm.at[c].at[pl.ds(s * ZR, ZR)])

    return agg_kernel(eidx, hs2)


def _out_body(a_ref, dis_ref, w_ref, b_ref, o_ref):
    a = jnp.concatenate([a_ref[0], a_ref[1]], axis=1) * dis_ref[...]
    y = jnp.dot(a, w_ref[...], preferred_element_type=jnp.float32) + b_ref[...]
    o_ref[...] = y * 0.5 * (1.0 + lax.erf(y * (2.0 ** -0.5)))


def _tc_out(acc, dis, W, b2):
    bm = 400
    return pl.pallas_call(
        _out_body,
        grid=(N // bm,),
        in_specs=[
            pl.BlockSpec((NC, bm, DC), lambda i: (0, i, 0)),
            pl.BlockSpec((bm, 1), lambda i: (i, 0)),
            pl.BlockSpec((D, D), lambda i: (0, 0)),
            pl.BlockSpec((1, D), lambda i: (0, 0)),
        ],
        out_specs=pl.BlockSpec((bm, D), lambda i: (i, 0)),
        out_shape=jax.ShapeDtypeStruct((N, D), jnp.float32),
    )(acc, dis, W, b2)


def kernel(x, edge_index, W, b, ln_gamma, ln_beta):
    rows = jnp.concatenate(
        [edge_index[0], jnp.full((EP - E,), PAD_ROW, jnp.int32)])
    cols = jnp.concatenate(
        [edge_index[1], jnp.full((EP - E,), PAD_COL, jnp.int32)])
    eidx = jnp.stack([rows.reshape(NB, G), cols.reshape(NB, G)], axis=1)
    x_pad = jnp.pad(x, ((0, NP - N), (0, 0)))

    hist = _sc_degree(rows.reshape(ND, GD))
    hs2, dis = _tc_norm_scale(x_pad, hist.T,
                              ln_gamma.reshape(1, D), ln_beta.reshape(1, D))
    acc = _sc_aggregate(eidx, hs2)
    return _tc_out(acc, dis, W, b.reshape(1, D))
